# baseline (speedup 1.0000x reference)
import functools

import jax
import jax.numpy as jnp
from jax import lax
from jax.experimental import pallas as pl
from jax.experimental.pallas import tpu as pltpu
from jax.experimental.pallas import tpu_sc as plsc


def _make_sc_kernel(batch, n_factors):
    info = plsc.get_sparse_core_info()
    nc, ns, lanes = info.num_cores, info.num_subcores, info.num_lanes
    nw = nc * ns
    assert batch % (8 * nw) == 0
    bpw = batch // nw
    mesh = plsc.VectorSubcoreMesh(core_axis_name="c", subcore_axis_name="s")

    @functools.partial(
        pl.kernel,
        out_type=jax.ShapeDtypeStruct((batch,), jnp.float32),
        mesh=mesh,
        compiler_params=pltpu.CompilerParams(
            needs_layout_passes=False, use_tc_tiling_on_sc=False
        ),
        scratch_types=[
            pltpu.VMEM((bpw,), jnp.int32),
            pltpu.VMEM((bpw,), jnp.int32),
            pltpu.VMEM((bpw, n_factors), jnp.float32),
            pltpu.VMEM((bpw, n_factors), jnp.float32),
            pltpu.VMEM((bpw,), jnp.float32),
            pltpu.VMEM((bpw,), jnp.float32),
            pltpu.VMEM((bpw,), jnp.float32),
            pltpu.SemaphoreType.DMA,
        ],
    )
    def mf_kernel(user_hbm, movie_hbm, uf_hbm, mf_hbm, ub_hbm, mb_hbm,
                  out_hbm, uidx, midx, urows, mrows, ubias, mbias, outv, sem):
        wid = lax.axis_index("s") * nc + lax.axis_index("c")
        base = wid * bpw
        pltpu.sync_copy(user_hbm.at[pl.ds(base, bpw)], uidx)
        pltpu.sync_copy(movie_hbm.at[pl.ds(base, bpw)], midx)
        c1 = pltpu.async_copy(uf_hbm.at[uidx], urows, sem)
        c2 = pltpu.async_copy(mf_hbm.at[midx], mrows, sem)
        c3 = pltpu.async_copy(ub_hbm.at[uidx], ubias, sem)
        c4 = pltpu.async_copy(mb_hbm.at[midx], mbias, sem)
        c1.wait()
        c2.wait()
        c3.wait()
        c4.wait()

        def group(g, _):
            rows = g * lanes + lax.iota(jnp.int32, lanes)
            acc = ubias[pl.ds(g * lanes, lanes)] + mbias[pl.ds(g * lanes, lanes)]
            for f in range(n_factors):
                cols = jnp.full((lanes,), f, jnp.int32)
                uv = plsc.load_gather(urows, [rows, cols])
                mv = plsc.load_gather(mrows, [rows, cols])
                acc = acc + uv * mv
            outv[pl.ds(g * lanes, lanes)] = acc
            return 0

        lax.fori_loop(0, bpw // lanes, group, 0)
        pltpu.sync_copy(outv, out_hbm.at[pl.ds(base, bpw)])

    return mf_kernel


def kernel(user, movie, user_factors, movie_factors, user_biases, movie_biases):
    batch = user.shape[0]
    n_factors = user_factors.shape[1]
    mf_kernel = _make_sc_kernel(batch, n_factors)
    return mf_kernel(
        user.astype(jnp.int32),
        movie.astype(jnp.int32),
        user_factors,
        movie_factors,
        user_biases.reshape(-1),
        movie_biases.reshape(-1),
    )
